# baseline (speedup 1.0000x reference)
import functools

import jax
import jax.numpy as jnp
from jax import lax
from jax.experimental import pallas as pl
from jax.experimental.pallas import tpu as pltpu, tpu_sc as plsc

NUM_ROWS = 100000
DIM = 64
BATCH = 4096
SEQ = 50
NC = 2
NS = 16
NW = NC * NS
ROWS_PER_W = BATCH // NW
NBUF = 4

_mesh = plsc.VectorSubcoreMesh(core_axis_name="c", subcore_axis_name="s")


@functools.partial(
    pl.kernel,
    out_type=jax.ShapeDtypeStruct((BATCH, SEQ, DIM), jnp.float32),
    mesh=_mesh,
    scratch_types=[
        pltpu.VMEM((ROWS_PER_W, SEQ), jnp.int32),
        pltpu.VMEM((NBUF, SEQ, DIM), jnp.float32),
        pltpu.SemaphoreType.DMA((NBUF,)),
        pltpu.SemaphoreType.DMA((NBUF,)),
    ],
    compiler_params=pltpu.CompilerParams(use_tc_tiling_on_sc=False),
)
def _gather_kernel(idx_hbm, table_hbm, out_hbm, idx_v, rows_v, gsem, osem):
    wid = lax.axis_index("s") * NC + lax.axis_index("c")
    base = wid * ROWS_PER_W
    pltpu.sync_copy(idx_hbm.at[pl.ds(base, ROWS_PER_W)], idx_v)

    for b in range(NBUF):
        pltpu.async_copy(table_hbm.at[idx_v.at[b]], rows_v.at[b], gsem.at[b])

    @pl.loop(0, ROWS_PER_W, step=NBUF)
    def _step(t):
        for b in range(NBUF):
            r = t + b
            pltpu.make_async_copy(
                table_hbm.at[idx_v.at[r]], rows_v.at[b], gsem.at[b]
            ).wait()
            out_cp = pltpu.async_copy(rows_v.at[b], out_hbm.at[base + r], osem.at[b])

            @pl.when(r + NBUF < ROWS_PER_W)
            def _refill():
                out_cp.wait()
                pltpu.async_copy(
                    table_hbm.at[idx_v.at[r + NBUF]], rows_v.at[b], gsem.at[b]
                )

    for b in range(NBUF):
        r = ROWS_PER_W - NBUF + b
        pltpu.make_async_copy(rows_v.at[b], out_hbm.at[base + r], osem.at[b]).wait()


def kernel(input, weight):
    return _gather_kernel(input.astype(jnp.int32), weight)
